# baseline (speedup 1.0000x reference)
import functools

import numpy as np

import jax
import jax.numpy as jnp
from jax.experimental import pallas as pl
from jax.experimental.pallas import tpu as pltpu


def _build_schedule(r):
    rows = []
    for i in range(r):
        order = [j for j in range(r) if j < i] + \
                [j for j in range(i + 1, r)] + [i]
        first_l1 = True
        n_l2 = 0
        for j in order:
            l1 = 1 if first_l1 else 2
            first_l1 = False
            l2 = 0
            fin_h = 0
            fin_out = 0
            if j < i:
                l2 = 1 if n_l2 == 0 else 2
                n_l2 += 1
            if j == i:
                fin_h = 1
                l2 = 1 if n_l2 == 0 else 2
                n_l2 += 1
                if i == r - 1:
                    fin_out = 1
            rows.append([i, j, l1, l2, fin_h, fin_out, r - 1, 0])
    for i in range(r - 1):
        for j in range(i + 1, r):
            fin_out = 1 if j == r - 1 else 0
            rows.append([i, j, 0, 2, 0, fin_out, i, 0])
    return np.asarray(rows, dtype=np.int32)


def _s1_body(x_ref, w1_ref, out_ref, *, n, pad):
    out_ref[pl.ds(0, n), :] = jnp.dot(x_ref[...], w1_ref[...],
                                      preferred_element_type=jnp.float32)
    if pad:
        out_ref[pl.ds(n, pad), :] = jnp.zeros((pad, out_ref.shape[1]),
                                              jnp.float32)


def _fused_body(sched_ref, adj_ref, s1_ref, b1_ref, w2_ref, b2_ref,
                out_ref, h2_ref, acc_ref, *, bk, n, r):
    t = pl.program_id(0)
    ti = sched_ref[t, 0]
    tj = sched_ref[t, 1]
    l1 = sched_ref[t, 2]
    l2 = sched_ref[t, 3]
    fin_h = sched_ref[t, 4]
    fin_out = sched_ref[t, 5]
    pad = r * bk - n

    if pad:
        @pl.when(tj == r - 1)
        def _():
            adj_ref[:, pl.ds(bk - pad, pad)] = jnp.zeros((bk, pad),
                                                         jnp.float32)

    ri = pl.ds(ti * bk, bk)
    rj = pl.ds(tj * bk, bk)

    @pl.when(l1 == 1)
    def _():
        h2_ref[ri, :] = jnp.dot(adj_ref[...].astype(jnp.bfloat16), s1_ref[rj, :].astype(jnp.bfloat16),
                                preferred_element_type=jnp.float32)

    @pl.when(l1 == 2)
    def _():
        h2_ref[ri, :] += jnp.dot(adj_ref[...].astype(jnp.bfloat16), s1_ref[rj, :].astype(jnp.bfloat16),
                                 preferred_element_type=jnp.float32)

    @pl.when(fin_h == 1)
    def _():
        h = jnp.maximum(h2_ref[ri, :] + b1_ref[...], 0.0)
        h2_ref[ri, :] = jnp.dot(h, w2_ref[...],
                                preferred_element_type=jnp.float32)

    if pad:
        @pl.when(jnp.logical_and(fin_h == 1, ti == r - 1))
        def _():
            h2_ref[pl.ds(n, pad), :] = jnp.zeros((pad, h2_ref.shape[1]),
                                                 jnp.float32)

    @pl.when(l2 == 1)
    def _():
        acc_ref[ri, :] = jnp.dot(adj_ref[...].astype(jnp.bfloat16), h2_ref[rj, :].astype(jnp.bfloat16),
                                 preferred_element_type=jnp.float32)

    @pl.when(l2 == 2)
    def _():
        acc_ref[ri, :] += jnp.dot(adj_ref[...].astype(jnp.bfloat16), h2_ref[rj, :].astype(jnp.bfloat16),
                                  preferred_element_type=jnp.float32)

    @pl.when(fin_out == 1)
    def _():
        v = acc_ref[ri, :] + b2_ref[...]
        m = jnp.max(v, axis=1, keepdims=True)
        s = v - m
        out_ref[...] = s - jnp.log(jnp.sum(jnp.exp(s), axis=1, keepdims=True))


def kernel(x, adj, W1, b1, W2, b2):
    n, _ = adj.shape
    d = W1.shape[1]
    bk = 2048 if n >= 2048 else 128
    r = -(-n // bk)
    pad = r * bk - n
    sched = jnp.asarray(_build_schedule(r))
    nsteps = sched.shape[0]

    s1 = pl.pallas_call(
        functools.partial(_s1_body, n=n, pad=pad),
        out_shape=jax.ShapeDtypeStruct((r * bk, d), jnp.float32),
    )(x, W1)

    body = functools.partial(_fused_body, bk=bk, n=n, r=r)

    grid_spec = pltpu.PrefetchScalarGridSpec(
        num_scalar_prefetch=1,
        grid=(nsteps,),
        in_specs=[
            pl.BlockSpec((bk, bk), lambda t, s: (s[t, 0], s[t, 1])),
            pl.BlockSpec((r * bk, d), lambda t, s: (0, 0)),
            pl.BlockSpec((1, d), lambda t, s: (0, 0)),
            pl.BlockSpec(W2.shape, lambda t, s: (0, 0)),
            pl.BlockSpec((1, d), lambda t, s: (0, 0)),
        ],
        out_specs=pl.BlockSpec((bk, d), lambda t, s: (s[t, 6], 0)),
        scratch_shapes=[
            pltpu.VMEM((r * bk, d), jnp.float32),
            pltpu.VMEM((r * bk, d), jnp.float32),
        ],
    )
    return pl.pallas_call(
        body,
        grid_spec=grid_spec,
        out_shape=jax.ShapeDtypeStruct((n, d), jnp.float32),
        compiler_params=pltpu.CompilerParams(
            dimension_semantics=("arbitrary",),
            vmem_limit_bytes=63 * 1024 * 1024,
        ),
    )(sched, adj, s1, b1.reshape(1, -1), W2, b2.reshape(1, -1))


# speedup vs baseline: 1.0305x; 1.0305x over previous
import functools

import numpy as np

import jax
import jax.numpy as jnp
from jax.experimental import pallas as pl
from jax.experimental.pallas import tpu as pltpu


def _build_schedule(r):
    rows = []
    for i in range(r):
        order = [j for j in range(r) if j < i] + \
                [j for j in range(i + 1, r)] + [i]
        first_l1 = True
        n_l2 = 0
        for j in order:
            l1 = 1 if first_l1 else 2
            first_l1 = False
            l2 = 0
            fin_h = 0
            fin_out = 0
            if j < i:
                l2 = 1 if n_l2 == 0 else 2
                n_l2 += 1
            if j == i:
                fin_h = 1
                l2 = 1 if n_l2 == 0 else 2
                n_l2 += 1
                if i == r - 1:
                    fin_out = 1
            rows.append([i, j, l1, l2, fin_h, fin_out, r - 1, 0])
    for i in range(r - 1):
        for j in range(i + 1, r):
            fin_out = 1 if j == r - 1 else 0
            rows.append([i, j, 0, 2, 0, fin_out, i, 0])
    return np.asarray(rows, dtype=np.int32)


def _fused_body(sched_ref, adj_ref, x_ref, w1_ref, b1_ref, w2_ref, b2_ref,
                out_ref, s1_ref, h2_ref, acc_ref, *, bk, n, r):
    t = pl.program_id(0)
    ti = sched_ref[t, 0]
    tj = sched_ref[t, 1]
    l1 = sched_ref[t, 2]
    l2 = sched_ref[t, 3]
    fin_h = sched_ref[t, 4]
    fin_out = sched_ref[t, 5]
    pad = r * bk - n

    @pl.when(t == 0)
    def _():
        s1_ref[pl.ds(0, n), :] = jnp.dot(x_ref[...], w1_ref[...],
                                         preferred_element_type=jnp.float32)
        if pad:
            s1_ref[pl.ds(n, pad), :] = jnp.zeros((pad, s1_ref.shape[1]),
                                                 jnp.float32)

    if pad:
        @pl.when(tj == r - 1)
        def _():
            adj_ref[:, pl.ds(bk - pad, pad)] = jnp.zeros((bk, pad),
                                                         jnp.float32)

    ri = pl.ds(ti * bk, bk)
    rj = pl.ds(tj * bk, bk)

    @pl.when(l1 == 1)
    def _():
        h2_ref[ri, :] = jnp.dot(adj_ref[...], s1_ref[rj, :],
                                preferred_element_type=jnp.float32)

    @pl.when(l1 == 2)
    def _():
        h2_ref[ri, :] += jnp.dot(adj_ref[...], s1_ref[rj, :],
                                 preferred_element_type=jnp.float32)

    @pl.when(fin_h == 1)
    def _():
        h = jnp.maximum(h2_ref[ri, :] + b1_ref[...], 0.0)
        h2_ref[ri, :] = jnp.dot(h, w2_ref[...],
                                preferred_element_type=jnp.float32)

    if pad:
        @pl.when(jnp.logical_and(fin_h == 1, ti == r - 1))
        def _():
            h2_ref[pl.ds(n, pad), :] = jnp.zeros((pad, h2_ref.shape[1]),
                                                 jnp.float32)

    @pl.when(l2 == 1)
    def _():
        acc_ref[ri, :] = jnp.dot(adj_ref[...], h2_ref[rj, :],
                                 preferred_element_type=jnp.float32)

    @pl.when(l2 == 2)
    def _():
        acc_ref[ri, :] += jnp.dot(adj_ref[...], h2_ref[rj, :],
                                  preferred_element_type=jnp.float32)

    @pl.when(fin_out == 1)
    def _():
        v = acc_ref[ri, :] + b2_ref[...]
        m = jnp.max(v, axis=1, keepdims=True)
        s = v - m
        out_ref[...] = s - jnp.log(jnp.sum(jnp.exp(s), axis=1, keepdims=True))


def kernel(x, adj, W1, b1, W2, b2):
    n, _ = adj.shape
    d = W1.shape[1]
    bk = 2048 if n >= 2048 else 128
    r = -(-n // bk)
    sched = jnp.asarray(_build_schedule(r))
    nsteps = sched.shape[0]

    body = functools.partial(_fused_body, bk=bk, n=n, r=r)

    grid_spec = pltpu.PrefetchScalarGridSpec(
        num_scalar_prefetch=1,
        grid=(nsteps,),
        in_specs=[
            pl.BlockSpec((bk, bk), lambda t, s: (s[t, 0], s[t, 1])),
            pl.BlockSpec((n, d), lambda t, s: (0, 0)),
            pl.BlockSpec(W1.shape, lambda t, s: (0, 0)),
            pl.BlockSpec((1, d), lambda t, s: (0, 0)),
            pl.BlockSpec(W2.shape, lambda t, s: (0, 0)),
            pl.BlockSpec((1, d), lambda t, s: (0, 0)),
        ],
        out_specs=pl.BlockSpec((bk, d), lambda t, s: (s[t, 6], 0)),
        scratch_shapes=[
            pltpu.VMEM((r * bk, d), jnp.float32),
            pltpu.VMEM((r * bk, d), jnp.float32),
            pltpu.VMEM((r * bk, d), jnp.float32),
        ],
    )
    return pl.pallas_call(
        body,
        grid_spec=grid_spec,
        out_shape=jax.ShapeDtypeStruct((n, d), jnp.float32),
        compiler_params=pltpu.CompilerParams(
            dimension_semantics=("arbitrary",),
            vmem_limit_bytes=63 * 1024 * 1024,
        ),
    )(sched, adj, x, W1, b1.reshape(1, -1), W2, b2.reshape(1, -1))
